# baseline (speedup 1.0000x reference)
import jax
import jax.numpy as jnp
from jax.experimental import pallas as pl

N_ROWS = 64
N_COLS = 32768
ROW_BLOCK = 64


def _hardmax_block(x_ref, o_ref):
    xb = x_ref[...]
    m = jnp.max(xb, axis=1, keepdims=True)
    iota = jax.lax.broadcasted_iota(jnp.int32, xb.shape, 1)
    idx = jnp.min(jnp.where(xb == m, iota, N_COLS), axis=1, keepdims=True)
    o_ref[...] = (iota == idx).astype(jnp.int32)


def kernel(x):
    return pl.pallas_call(
        _hardmax_block,
        grid=(N_ROWS // ROW_BLOCK,),
        in_specs=[pl.BlockSpec((ROW_BLOCK, N_COLS), lambda i: (i, 0))],
        out_specs=pl.BlockSpec((ROW_BLOCK, N_COLS), lambda i: (i, 0)),
        out_shape=jax.ShapeDtypeStruct((N_ROWS, N_COLS), jnp.int32),
    )(x)


# speedup vs baseline: 1.2519x; 1.2519x over previous
import jax
import jax.numpy as jnp
from jax.experimental import pallas as pl

N_ROWS = 64
N_COLS = 32768
ROW_BLOCK = 32


def _hardmax_block(x_ref, o_ref):
    xb = x_ref[...]
    m = jnp.max(xb, axis=1, keepdims=True)
    iota = jax.lax.broadcasted_iota(jnp.int32, xb.shape, 1)
    idx = jnp.min(jnp.where(xb == m, iota, N_COLS), axis=1, keepdims=True)
    o_ref[...] = (iota == idx).astype(jnp.int32)


def kernel(x):
    return pl.pallas_call(
        _hardmax_block,
        grid=(N_ROWS // ROW_BLOCK,),
        in_specs=[pl.BlockSpec((ROW_BLOCK, N_COLS), lambda i: (i, 0))],
        out_specs=pl.BlockSpec((ROW_BLOCK, N_COLS), lambda i: (i, 0)),
        out_shape=jax.ShapeDtypeStruct((N_ROWS, N_COLS), jnp.int32),
    )(x)


# speedup vs baseline: 1.3004x; 1.0388x over previous
import jax
import jax.numpy as jnp
from jax.experimental import pallas as pl

N_ROWS = 64
N_COLS = 32768
ROW_BLOCK = 32


def _hardmax_block(x_ref, o_ref):
    xb = x_ref[...]
    idx = jnp.argmax(xb, axis=1)
    iota = jax.lax.broadcasted_iota(jnp.int32, xb.shape, 1)
    o_ref[...] = (iota == idx[:, None]).astype(jnp.int32)


def kernel(x):
    return pl.pallas_call(
        _hardmax_block,
        grid=(N_ROWS // ROW_BLOCK,),
        in_specs=[pl.BlockSpec((ROW_BLOCK, N_COLS), lambda i: (i, 0))],
        out_specs=pl.BlockSpec((ROW_BLOCK, N_COLS), lambda i: (i, 0)),
        out_shape=jax.ShapeDtypeStruct((N_ROWS, N_COLS), jnp.int32),
    )(x)
